# baseline (speedup 1.0000x reference)
import functools

import jax
import jax.numpy as jnp
from jax import lax
from jax.experimental import pallas as pl
from jax.experimental.pallas import tpu as pltpu
from jax.experimental.pallas import tpu_sc as plsc

D_MODEL = 768
LANES = 16
NVEC = D_MODEL // LANES
N_WORKERS = 32
CHUNK = 16
NBUF = 5


def _rsqrt(x):
    xi = lax.bitcast_convert_type(x, jnp.int32)
    yi = jnp.full((LANES,), 0x5F3759DF, jnp.int32) - (xi >> 1)
    y = lax.bitcast_convert_type(yi, jnp.float32)
    for _ in range(3):
        y = y * (1.5 - 0.5 * x * y * y)
    return y


_GATHER_DNUMS = lax.GatherDimensionNumbers(
    offset_dims=(), collapsed_slice_dims=(0,), start_index_map=(0,))


def _permute(v, idx):
    return lax.gather(v, idx[:, None], _GATHER_DNUMS, slice_sizes=(1,),
                      mode=lax.GatherScatterMode.PROMISE_IN_BOUNDS)


def _lane_sum(v, perms):
    for idx in perms:
        v = v + _permute(v, idx)
    return v


def _make_sc_kernel(batch, seq):
    n_tokens = batch * seq
    sblock = seq // N_WORKERS
    nq = sblock // CHUNK
    mesh = plsc.VectorSubcoreMesh(core_axis_name="c", subcore_axis_name="s")

    @functools.partial(
        pl.kernel,
        mesh=mesh,
        out_type=jax.ShapeDtypeStruct((n_tokens, D_MODEL), jnp.float32),
        scratch_types=[
            pltpu.VMEM((batch, nq, CHUNK), jnp.int32),
            pltpu.VMEM((NBUF, CHUNK, D_MODEL), jnp.float32),
            pltpu.VMEM((sblock, D_MODEL), jnp.float32),
            pltpu.VMEM((D_MODEL,), jnp.float32),
            pltpu.VMEM((D_MODEL,), jnp.float32),
            pltpu.VMEM((CHUNK, LANES), jnp.float32),
            pltpu.VMEM((CHUNK, LANES), jnp.float32),
            pltpu.SemaphoreType.DMA((NBUF,)),
            pltpu.SemaphoreType.DMA((NBUF,)),
        ],
    )
    def k(ids_hbm, w_hbm, pos_hbm, gamma_hbm, beta_hbm, out_hbm,
          idx_v, rows_v, pos_v, g_v, b_v, mean_v, inv_v, sg, so):
        nc = 2
        wid = lax.axis_index("s") * nc + lax.axis_index("c")
        sbase = wid * sblock

        pltpu.sync_copy(ids_hbm.at[wid], idx_v)

        def gather(b, q, buf):
            return pltpu.async_copy(
                w_hbm.at[idx_v.at[b, q]], rows_v.at[buf], sg.at[buf])

        def out_copy(b, q, buf):
            tok = b * seq + sbase + q * CHUNK
            return pltpu.async_copy(
                rows_v.at[buf], out_hbm.at[pl.ds(tok, CHUNK)], so.at[buf])

        perms = [lax.iota(jnp.int32, LANES) ^ s for s in (8, 4, 2, 1)]

        def make_stats_body(buf, poff):
            def stats_body(r):
                def sum_range(lo, hi):
                    if hi - lo == 1:
                        sl = pl.ds(lo * LANES, LANES)
                        e = rows_v[buf, r, sl] + pos_v[poff + r, sl]
                        rows_v[buf, r, sl] = e
                        return e, e * e
                    mid = (lo + hi) // 2
                    a1, a2 = sum_range(lo, mid)
                    b1, b2 = sum_range(mid, hi)
                    return a1 + b1, a2 + b2

                BLK = 8
                s1, s2 = sum_range(0, BLK)
                for blk in range(1, NVEC // BLK):
                    t1, t2 = sum_range(blk * BLK, (blk + 1) * BLK)
                    s1 = s1 + t1
                    s2 = s2 + t2
                mean = _lane_sum(s1, perms) * (1.0 / D_MODEL)
                m2 = _lane_sum(s2, perms) * (1.0 / D_MODEL)
                var = jnp.maximum(m2 - mean * mean, 0.0)
                mean_v[r, :] = mean
                inv_v[r, :] = _rsqrt(var + 1e-12)
            return stats_body

        NJB = 4
        jblk = NVEC // NJB

        def make_norm_body(buf, jb, gregs, bregs):
            def norm_body(r):
                mean = mean_v[r, :]
                inv = inv_v[r, :]
                for j in range(jblk):
                    sl = pl.ds((jb * jblk + j) * LANES, LANES)
                    e = rows_v[buf, r, sl]
                    rows_v[buf, r, sl] = (e - mean) * inv * gregs[j] + bregs[j]
            return norm_body

        def compute(buf, poff):
            plsc.parallel_loop(0, CHUNK, unroll=2)(make_stats_body(buf, poff))
            for jb in range(NJB):
                gregs = [g_v[pl.ds((jb * jblk + j) * LANES, LANES)]
                         for j in range(jblk)]
                bregs = [b_v[pl.ds((jb * jblk + j) * LANES, LANES)]
                         for j in range(jblk)]
                plsc.parallel_loop(0, CHUNK, unroll=1)(
                    make_norm_body(buf, jb, gregs, bregs))

        n_chunks = batch * nq

        gather(0, 0, 0)
        gather(0, 1, 1)
        gather(0, 2, 2)
        pltpu.sync_copy(pos_hbm.at[pl.ds(sbase, sblock)], pos_v)
        pltpu.sync_copy(gamma_hbm, g_v)
        pltpu.sync_copy(beta_hbm, b_v)

        DIST = 3

        def chunk_body(p, _):
            b = p // nq
            q = p % nq
            buf = p % NBUF
            pltpu.make_async_copy(
                w_hbm.at[idx_v.at[b, q]], rows_v.at[buf], sg.at[buf]).wait()
            compute(buf, q * CHUNK)
            out_copy(b, q, buf)

            p2 = p + DIST
            b2 = p2 // nq
            q2 = p2 % nq
            buf2 = p2 % NBUF

            @pl.when(jnp.logical_and(p2 < n_chunks, p >= NBUF - DIST))
            def _():
                pltpu.make_async_copy(
                    rows_v.at[buf2],
                    out_hbm.at[pl.ds(q2 * CHUNK, CHUNK)],
                    so.at[buf2]).wait()
                gather(b2, q2, buf2)

            @pl.when(jnp.logical_and(p2 < n_chunks, p < NBUF - DIST))
            def _():
                gather(b2, q2, buf2)
            return 0

        lax.fori_loop(0, n_chunks, chunk_body, 0)
        for k in range(NBUF):
            pltpu.make_async_copy(
                rows_v.at[k],
                out_hbm.at[pl.ds(0, CHUNK)], so.at[k]).wait()

    return k


@jax.jit
def kernel(input_ids, W, pos, gamma, beta):
    batch, seq = input_ids.shape
    sblock = seq // N_WORKERS
    nq = sblock // CHUNK
    ids = input_ids.reshape(batch, N_WORKERS, nq, CHUNK)
    ids = ids.transpose(1, 0, 2, 3).astype(jnp.int32)
    sc = _make_sc_kernel(batch, seq)
    out = sc(ids, W, pos[0, :seq], gamma, beta)
    return out.reshape(batch, seq, D_MODEL)


# speedup vs baseline: 1.0119x; 1.0119x over previous
import functools

import jax
import jax.numpy as jnp
from jax import lax
from jax.experimental import pallas as pl
from jax.experimental.pallas import tpu as pltpu
from jax.experimental.pallas import tpu_sc as plsc

D_MODEL = 768
LANES = 16
NVEC = D_MODEL // LANES
N_WORKERS = 32
CHUNK = 16
NBUF = 4


def _rsqrt(x):
    xi = lax.bitcast_convert_type(x, jnp.int32)
    yi = jnp.full((LANES,), 0x5F3759DF, jnp.int32) - (xi >> 1)
    y = lax.bitcast_convert_type(yi, jnp.float32)
    for _ in range(3):
        y = y * (1.5 - 0.5 * x * y * y)
    return y


_GATHER_DNUMS = lax.GatherDimensionNumbers(
    offset_dims=(), collapsed_slice_dims=(0,), start_index_map=(0,))


def _permute(v, idx):
    return lax.gather(v, idx[:, None], _GATHER_DNUMS, slice_sizes=(1,),
                      mode=lax.GatherScatterMode.PROMISE_IN_BOUNDS)


def _lane_sum(v, perms):
    for idx in perms:
        v = v + _permute(v, idx)
    return v


def _make_sc_kernel(batch, seq):
    n_tokens = batch * seq
    sblock = seq // N_WORKERS
    nq = sblock // CHUNK
    mesh = plsc.VectorSubcoreMesh(core_axis_name="c", subcore_axis_name="s")

    @functools.partial(
        pl.kernel,
        mesh=mesh,
        out_type=jax.ShapeDtypeStruct((n_tokens, D_MODEL), jnp.float32),
        scratch_types=[
            pltpu.VMEM((batch, nq, CHUNK), jnp.int32),
            pltpu.VMEM((NBUF, CHUNK, D_MODEL), jnp.float32),
            pltpu.VMEM((sblock, D_MODEL), jnp.float32),
            pltpu.VMEM((D_MODEL,), jnp.float32),
            pltpu.VMEM((D_MODEL,), jnp.float32),
            pltpu.VMEM((CHUNK, LANES), jnp.float32),
            pltpu.VMEM((CHUNK, LANES), jnp.float32),
            pltpu.SemaphoreType.DMA((NBUF,)),
            pltpu.SemaphoreType.DMA((NBUF,)),
        ],
    )
    def k(ids_hbm, w_hbm, pos_hbm, gamma_hbm, beta_hbm, out_hbm,
          idx_v, rows_v, pos_v, g_v, b_v, mean_v, inv_v, sg, so):
        nc = 2
        wid = lax.axis_index("s") * nc + lax.axis_index("c")
        sbase = wid * sblock

        pltpu.sync_copy(ids_hbm.at[wid], idx_v)

        def gather(b, q, buf):
            return pltpu.async_copy(
                w_hbm.at[idx_v.at[b, q]], rows_v.at[buf], sg.at[buf])

        def out_copy(b, q, buf):
            tok = b * seq + sbase + q * CHUNK
            return pltpu.async_copy(
                rows_v.at[buf], out_hbm.at[pl.ds(tok, CHUNK)], so.at[buf])

        perms = [lax.iota(jnp.int32, LANES) ^ s for s in (8, 4, 2, 1)]

        def make_stats_body(buf, poff):
            def stats_body(r):
                def sum_range(lo, hi):
                    if hi - lo == 1:
                        sl = pl.ds(lo * LANES, LANES)
                        e = rows_v[buf, r, sl] + pos_v[poff + r, sl]
                        rows_v[buf, r, sl] = e
                        return e, e * e
                    mid = (lo + hi) // 2
                    a1, a2 = sum_range(lo, mid)
                    b1, b2 = sum_range(mid, hi)
                    return a1 + b1, a2 + b2

                BLK = 8
                s1, s2 = sum_range(0, BLK)
                for blk in range(1, NVEC // BLK):
                    t1, t2 = sum_range(blk * BLK, (blk + 1) * BLK)
                    s1 = s1 + t1
                    s2 = s2 + t2
                mean = _lane_sum(s1, perms) * (1.0 / D_MODEL)
                m2 = _lane_sum(s2, perms) * (1.0 / D_MODEL)
                var = jnp.maximum(m2 - mean * mean, 0.0)
                mean_v[r, :] = mean
                inv_v[r, :] = _rsqrt(var + 1e-12)
            return stats_body

        NJB = 4
        jblk = NVEC // NJB

        def make_norm_body(buf, jb, gregs, bregs):
            def norm_body(r):
                mean = mean_v[r, :]
                inv = inv_v[r, :]
                for j in range(jblk):
                    sl = pl.ds((jb * jblk + j) * LANES, LANES)
                    e = rows_v[buf, r, sl]
                    rows_v[buf, r, sl] = (e - mean) * inv * gregs[j] + bregs[j]
            return norm_body

        def compute(buf, poff):
            plsc.parallel_loop(0, CHUNK, unroll=2)(make_stats_body(buf, poff))
            for jb in range(NJB):
                gregs = [g_v[pl.ds((jb * jblk + j) * LANES, LANES)]
                         for j in range(jblk)]
                bregs = [b_v[pl.ds((jb * jblk + j) * LANES, LANES)]
                         for j in range(jblk)]
                plsc.parallel_loop(0, CHUNK, unroll=1)(
                    make_norm_body(buf, jb, gregs, bregs))

        n_chunks = batch * nq

        gather(0, 0, 0)
        gather(0, 1, 1)
        pltpu.sync_copy(pos_hbm.at[pl.ds(sbase, sblock)], pos_v)
        pltpu.sync_copy(gamma_hbm, g_v)
        pltpu.sync_copy(beta_hbm, b_v)

        DIST = 2

        def chunk_body(p, _):
            b = p // nq
            q = p % nq
            buf = p % NBUF
            pltpu.make_async_copy(
                w_hbm.at[idx_v.at[b, q]], rows_v.at[buf], sg.at[buf]).wait()
            compute(buf, q * CHUNK)
            out_copy(b, q, buf)

            p2 = p + DIST
            b2 = p2 // nq
            q2 = p2 % nq
            buf2 = p2 % NBUF

            @pl.when(jnp.logical_and(p2 < n_chunks, p >= NBUF - DIST))
            def _():
                pltpu.make_async_copy(
                    rows_v.at[buf2],
                    out_hbm.at[pl.ds(q2 * CHUNK, CHUNK)],
                    so.at[buf2]).wait()
                gather(b2, q2, buf2)

            @pl.when(jnp.logical_and(p2 < n_chunks, p < NBUF - DIST))
            def _():
                gather(b2, q2, buf2)
            return 0

        lax.fori_loop(0, n_chunks, chunk_body, 0)
        for k in range(NBUF):
            pltpu.make_async_copy(
                rows_v.at[k],
                out_hbm.at[pl.ds(0, CHUNK)], so.at[k]).wait()

    return k


@jax.jit
def kernel(input_ids, W, pos, gamma, beta):
    batch, seq = input_ids.shape
    sblock = seq // N_WORKERS
    nq = sblock // CHUNK
    ids = input_ids.reshape(batch, N_WORKERS, nq, CHUNK)
    ids = ids.transpose(1, 0, 2, 3).astype(jnp.int32)
    sc = _make_sc_kernel(batch, seq)
    out = sc(ids, W, pos[0, :seq], gamma, beta)
    return out.reshape(batch, seq, D_MODEL)


# speedup vs baseline: 1.0187x; 1.0066x over previous
import functools

import jax
import jax.numpy as jnp
from jax import lax
from jax.experimental import pallas as pl
from jax.experimental.pallas import tpu as pltpu
from jax.experimental.pallas import tpu_sc as plsc

D_MODEL = 768
LANES = 16
NVEC = D_MODEL // LANES
N_WORKERS = 32
CHUNK = 16
NBUF = 4


def _rsqrt(x):
    xi = lax.bitcast_convert_type(x, jnp.int32)
    yi = jnp.full((LANES,), 0x5F3759DF, jnp.int32) - (xi >> 1)
    y = lax.bitcast_convert_type(yi, jnp.float32)
    for _ in range(3):
        y = y * (1.5 - 0.5 * x * y * y)
    return y


_GATHER_DNUMS = lax.GatherDimensionNumbers(
    offset_dims=(), collapsed_slice_dims=(0,), start_index_map=(0,))


def _permute(v, idx):
    return lax.gather(v, idx[:, None], _GATHER_DNUMS, slice_sizes=(1,),
                      mode=lax.GatherScatterMode.PROMISE_IN_BOUNDS)


def _lane_sum(v, perms):
    for idx in perms:
        v = v + _permute(v, idx)
    return v


def _make_sc_kernel(batch, seq):
    n_tokens = batch * seq
    sblock = seq // N_WORKERS
    nq = sblock // CHUNK
    mesh = plsc.VectorSubcoreMesh(core_axis_name="c", subcore_axis_name="s")

    @functools.partial(
        pl.kernel,
        mesh=mesh,
        out_type=jax.ShapeDtypeStruct((n_tokens, D_MODEL), jnp.float32),
        scratch_types=[
            pltpu.VMEM((batch, nq, CHUNK), jnp.int32),
            pltpu.VMEM((NBUF, CHUNK, D_MODEL), jnp.float32),
            pltpu.VMEM((sblock, D_MODEL), jnp.float32),
            pltpu.VMEM((D_MODEL,), jnp.float32),
            pltpu.VMEM((D_MODEL,), jnp.float32),
            pltpu.VMEM((CHUNK, LANES), jnp.float32),
            pltpu.VMEM((CHUNK, LANES), jnp.float32),
            pltpu.SemaphoreType.DMA((NBUF,)),
            pltpu.SemaphoreType.DMA((NBUF,)),
            pltpu.SemaphoreType.DMA,
        ],
    )
    def k(ids_hbm, w_hbm, pos_hbm, gamma_hbm, beta_hbm, out_hbm,
          idx_v, rows_v, pos_v, g_v, b_v, mean_v, inv_v, sg, so, sp):
        nc = 2
        wid = lax.axis_index("s") * nc + lax.axis_index("c")
        sbase = wid * sblock

        pltpu.sync_copy(ids_hbm.at[wid], idx_v)

        def gather(b, q, buf):
            return pltpu.async_copy(
                w_hbm.at[idx_v.at[b, q]], rows_v.at[buf], sg.at[buf])

        def out_copy(b, q, buf):
            tok = b * seq + sbase + q * CHUNK
            return pltpu.async_copy(
                rows_v.at[buf], out_hbm.at[pl.ds(tok, CHUNK)], so.at[buf])

        perms = [lax.iota(jnp.int32, LANES) ^ s for s in (8, 4, 2, 1)]

        def make_stats_body(buf, poff):
            def stats_body(r):
                def sum_range(lo, hi):
                    if hi - lo == 1:
                        sl = pl.ds(lo * LANES, LANES)
                        e = rows_v[buf, r, sl] + pos_v[poff + r, sl]
                        rows_v[buf, r, sl] = e
                        return e, e * e
                    mid = (lo + hi) // 2
                    a1, a2 = sum_range(lo, mid)
                    b1, b2 = sum_range(mid, hi)
                    return a1 + b1, a2 + b2

                BLK = 8
                s1, s2 = sum_range(0, BLK)
                for blk in range(1, NVEC // BLK):
                    t1, t2 = sum_range(blk * BLK, (blk + 1) * BLK)
                    s1 = s1 + t1
                    s2 = s2 + t2
                mean = _lane_sum(s1, perms) * (1.0 / D_MODEL)
                m2 = _lane_sum(s2, perms) * (1.0 / D_MODEL)
                var = jnp.maximum(m2 - mean * mean, 0.0)
                mean_v[r, :] = mean
                inv_v[r, :] = _rsqrt(var + 1e-12)
            return stats_body

        NJB = 4
        jblk = NVEC // NJB

        def make_norm_body(buf, jb, gregs, bregs):
            def norm_body(r):
                mean = mean_v[r, :]
                inv = inv_v[r, :]
                for j in range(jblk):
                    sl = pl.ds((jb * jblk + j) * LANES, LANES)
                    e = rows_v[buf, r, sl]
                    rows_v[buf, r, sl] = (e - mean) * inv * gregs[j] + bregs[j]
            return norm_body

        def compute(buf, poff):
            plsc.parallel_loop(0, CHUNK, unroll=2)(make_stats_body(buf, poff))
            for jb in range(NJB):
                gregs = [g_v[pl.ds((jb * jblk + j) * LANES, LANES)]
                         for j in range(jblk)]
                bregs = [b_v[pl.ds((jb * jblk + j) * LANES, LANES)]
                         for j in range(jblk)]
                plsc.parallel_loop(0, CHUNK, unroll=1)(
                    make_norm_body(buf, jb, gregs, bregs))

        n_chunks = batch * nq

        gather(0, 0, 0)
        gather(0, 1, 1)
        half = sblock // 2
        pltpu.async_copy(pos_hbm.at[pl.ds(sbase + half, half)],
                         pos_v.at[pl.ds(half, half)], sp)
        pltpu.sync_copy(pos_hbm.at[pl.ds(sbase, half)],
                        pos_v.at[pl.ds(0, half)])
        pltpu.sync_copy(gamma_hbm, g_v)
        pltpu.sync_copy(beta_hbm, b_v)

        DIST = 2

        def chunk_body(p, _):
            b = p // nq
            q = p % nq
            buf = p % NBUF
            pltpu.make_async_copy(
                w_hbm.at[idx_v.at[b, q]], rows_v.at[buf], sg.at[buf]).wait()

            @pl.when(p == nq // 2)
            def _():
                half = sblock // 2
                pltpu.make_async_copy(
                    pos_hbm.at[pl.ds(sbase + half, half)],
                    pos_v.at[pl.ds(half, half)], sp).wait()
            compute(buf, q * CHUNK)
            out_copy(b, q, buf)

            p2 = p + DIST
            b2 = p2 // nq
            q2 = p2 % nq
            buf2 = p2 % NBUF

            @pl.when(jnp.logical_and(p2 < n_chunks, p >= NBUF - DIST))
            def _():
                pltpu.make_async_copy(
                    rows_v.at[buf2],
                    out_hbm.at[pl.ds(q2 * CHUNK, CHUNK)],
                    so.at[buf2]).wait()
                gather(b2, q2, buf2)

            @pl.when(jnp.logical_and(p2 < n_chunks, p < NBUF - DIST))
            def _():
                gather(b2, q2, buf2)
            return 0

        lax.fori_loop(0, n_chunks, chunk_body, 0)
        for k in range(NBUF):
            pltpu.make_async_copy(
                rows_v.at[k],
                out_hbm.at[pl.ds(0, CHUNK)], so.at[k]).wait()

    return k


@jax.jit
def kernel(input_ids, W, pos, gamma, beta):
    batch, seq = input_ids.shape
    sblock = seq // N_WORKERS
    nq = sblock // CHUNK
    ids = input_ids.reshape(batch, N_WORKERS, nq, CHUNK)
    ids = ids.transpose(1, 0, 2, 3).astype(jnp.int32)
    sc = _make_sc_kernel(batch, seq)
    out = sc(ids, W, pos[0, :seq], gamma, beta)
    return out.reshape(batch, seq, D_MODEL)
